# baseline (speedup 1.0000x reference)
import functools

import jax
import jax.numpy as jnp
from jax import lax
from jax.experimental import pallas as pl
from jax.experimental.pallas import tpu as pltpu
from jax.experimental.pallas import tpu_sc as plsc

EMBED = 64
NUM_CORES = 2
NUM_SUBCORES = 16
NUM_WORKERS = NUM_CORES * NUM_SUBCORES
CHUNK = 128


def kernel(author_idx, publisher_idx, year_idx, author_table,
           publisher_table, year_table):
    batch = author_idx.shape[0]
    b_per_w = batch // NUM_WORKERS
    n_chunks = b_per_w // CHUNK

    a_idx = author_idx.astype(jnp.int32)
    p_idx = publisher_idx.astype(jnp.int32)
    y_idx = year_idx.astype(jnp.int32)

    mesh = plsc.VectorSubcoreMesh(core_axis_name="c", subcore_axis_name="s")

    @functools.partial(
        pl.kernel,
        mesh=mesh,
        out_type=jax.ShapeDtypeStruct((batch, 3, EMBED), jnp.float32),
        scratch_types=[
            pltpu.VMEM((b_per_w,), jnp.int32),
            pltpu.VMEM((b_per_w,), jnp.int32),
            pltpu.VMEM((b_per_w,), jnp.int32),
            pltpu.VMEM((b_per_w, EMBED), jnp.float32),
            pltpu.VMEM((b_per_w, EMBED), jnp.float32),
            pltpu.VMEM((b_per_w, EMBED), jnp.float32),
            pltpu.SemaphoreType.DMA,
            pltpu.SemaphoreType.DMA,
            pltpu.SemaphoreType.DMA,
        ],
        compiler_params=pltpu.CompilerParams(use_tc_tiling_on_sc=False),
    )
    def sc_gather3(a_hbm, p_hbm, y_hbm, ai, pi, yi, out,
                   ai_v, pi_v, yi_v, ar_v, pr_v, yr_v,
                   sem_a, sem_p, sem_y):
        wid = lax.axis_index("s") * NUM_CORES + lax.axis_index("c")
        base = wid * b_per_w
        pltpu.sync_copy(ai.at[pl.ds(base, b_per_w)], ai_v)
        pltpu.sync_copy(pi.at[pl.ds(base, b_per_w)], pi_v)
        pltpu.sync_copy(yi.at[pl.ds(base, b_per_w)], yi_v)
        ca, cp, cy = [], [], []
        for j in range(n_chunks):
            rows = pl.ds(j * CHUNK, CHUNK)
            ca.append(pltpu.async_copy(
                a_hbm.at[ai_v.at[rows]], ar_v.at[rows], sem_a))
            cp.append(pltpu.async_copy(
                p_hbm.at[pi_v.at[rows]], pr_v.at[rows], sem_p))
            cy.append(pltpu.async_copy(
                y_hbm.at[yi_v.at[rows]], yr_v.at[rows], sem_y))
        dst = pl.ds(base, b_per_w)
        for c in ca:
            c.wait()
        wa = pltpu.async_copy(ar_v, out.at[dst, 0], sem_a)
        for c in cp:
            c.wait()
        wp = pltpu.async_copy(pr_v, out.at[dst, 1], sem_p)
        for c in cy:
            c.wait()
        wy = pltpu.async_copy(yr_v, out.at[dst, 2], sem_y)
        wa.wait()
        wp.wait()
        wy.wait()

    out3 = sc_gather3(author_table, publisher_table, year_table,
                      a_idx, p_idx, y_idx)
    return out3.reshape(batch, 3 * EMBED)


# speedup vs baseline: 1.0100x; 1.0100x over previous
import functools

import jax
import jax.numpy as jnp
from jax import lax
from jax.experimental import pallas as pl
from jax.experimental.pallas import tpu as pltpu
from jax.experimental.pallas import tpu_sc as plsc

EMBED = 64
NUM_CORES = 2
NUM_SUBCORES = 16
NUM_WORKERS = NUM_CORES * NUM_SUBCORES
CHUNK = 128


def _sc_gather_half(a_idx, p_idx, y_idx, author_table, publisher_table,
                    year_table):
    batch = a_idx.shape[0]
    b_per_w = batch // NUM_WORKERS
    n_chunks = b_per_w // CHUNK

    mesh = plsc.VectorSubcoreMesh(core_axis_name="c", subcore_axis_name="s")

    @functools.partial(
        pl.kernel,
        mesh=mesh,
        out_type=jax.ShapeDtypeStruct((batch, 3 * EMBED), jnp.float32),
        scratch_types=[
            pltpu.VMEM((b_per_w,), jnp.int32),
            pltpu.VMEM((b_per_w,), jnp.int32),
            pltpu.VMEM((b_per_w,), jnp.int32),
            pltpu.VMEM((b_per_w, EMBED), jnp.float32),
            pltpu.VMEM((b_per_w, EMBED), jnp.float32),
            pltpu.VMEM((b_per_w, EMBED), jnp.float32),
            pltpu.SemaphoreType.DMA,
            pltpu.SemaphoreType.DMA,
            pltpu.SemaphoreType.DMA,
        ],
        compiler_params=pltpu.CompilerParams(use_tc_tiling_on_sc=False),
    )
    def sc_gather3(a_hbm, p_hbm, y_hbm, ai, pi, yi, out,
                   ai_v, pi_v, yi_v, ar_v, pr_v, yr_v,
                   sem_a, sem_p, sem_y):
        wid = lax.axis_index("s") * NUM_CORES + lax.axis_index("c")
        base = wid * b_per_w
        pltpu.sync_copy(ai.at[pl.ds(base, b_per_w)], ai_v)
        pltpu.sync_copy(pi.at[pl.ds(base, b_per_w)], pi_v)
        pltpu.sync_copy(yi.at[pl.ds(base, b_per_w)], yi_v)
        ca, cp, cy = [], [], []
        for j in range(n_chunks):
            rows = pl.ds(j * CHUNK, CHUNK)
            ca.append(pltpu.async_copy(
                a_hbm.at[ai_v.at[rows]], ar_v.at[rows], sem_a))
            cp.append(pltpu.async_copy(
                p_hbm.at[pi_v.at[rows]], pr_v.at[rows], sem_p))
            cy.append(pltpu.async_copy(
                y_hbm.at[yi_v.at[rows]], yr_v.at[rows], sem_y))
        dst = pl.ds(base, b_per_w)
        for c in ca:
            c.wait()
        wa = pltpu.async_copy(ar_v, out.at[dst, pl.ds(0, EMBED)], sem_a)
        for c in cp:
            c.wait()
        wp = pltpu.async_copy(pr_v, out.at[dst, pl.ds(EMBED, EMBED)], sem_p)
        for c in cy:
            c.wait()
        wy = pltpu.async_copy(yr_v, out.at[dst, pl.ds(2 * EMBED, EMBED)],
                              sem_y)
        wa.wait()
        wp.wait()
        wy.wait()

    return sc_gather3(author_table, publisher_table, year_table,
                      a_idx, p_idx, y_idx)


def kernel(author_idx, publisher_idx, year_idx, author_table,
           publisher_table, year_table):
    batch = author_idx.shape[0]
    half = batch // 2

    a_idx = author_idx.astype(jnp.int32)
    p_idx = publisher_idx.astype(jnp.int32)
    y_idx = year_idx.astype(jnp.int32)

    tables = (author_table, publisher_table, year_table)
    out1 = _sc_gather_half(a_idx[:half], p_idx[:half], y_idx[:half], *tables)
    out2 = _sc_gather_half(a_idx[half:], p_idx[half:], y_idx[half:], *tables)
    return jnp.concatenate((out1, out2), axis=0)


# speedup vs baseline: 1.2148x; 1.2028x over previous
import functools

import jax
import jax.numpy as jnp
from jax import lax
from jax.experimental import pallas as pl
from jax.experimental.pallas import tpu as pltpu
from jax.experimental.pallas import tpu_sc as plsc

EMBED = 64
NUM_CORES = 2
NUM_SUBCORES = 16
NUM_WORKERS = NUM_CORES * NUM_SUBCORES
CHUNK = 128


def _sc_gather_half(a_idx, p_idx, y_idx, author_table, publisher_table,
                    year_table):
    batch = a_idx.shape[0]
    b_per_w = batch // NUM_WORKERS
    n_chunks = b_per_w // CHUNK

    mesh = plsc.VectorSubcoreMesh(core_axis_name="c", subcore_axis_name="s")

    @functools.partial(
        pl.kernel,
        mesh=mesh,
        out_type=jax.ShapeDtypeStruct((batch, 3 * EMBED), jnp.float32),
        scratch_types=[
            pltpu.VMEM((b_per_w,), jnp.int32),
            pltpu.VMEM((b_per_w,), jnp.int32),
            pltpu.VMEM((b_per_w,), jnp.int32),
            pltpu.VMEM((b_per_w, EMBED), jnp.float32),
            pltpu.VMEM((b_per_w, EMBED), jnp.float32),
            pltpu.VMEM((b_per_w, EMBED), jnp.float32),
            pltpu.SemaphoreType.DMA,
            pltpu.SemaphoreType.DMA,
            pltpu.SemaphoreType.DMA,
        ],
        compiler_params=pltpu.CompilerParams(use_tc_tiling_on_sc=False),
    )
    def sc_gather3(a_hbm, p_hbm, y_hbm, ai, pi, yi, out,
                   ai_v, pi_v, yi_v, ar_v, pr_v, yr_v,
                   sem_a, sem_p, sem_y):
        wid = lax.axis_index("s") * NUM_CORES + lax.axis_index("c")
        base = wid * b_per_w
        la = pltpu.async_copy(ai.at[pl.ds(base, b_per_w)], ai_v, sem_a)
        lp = pltpu.async_copy(pi.at[pl.ds(base, b_per_w)], pi_v, sem_p)
        ly = pltpu.async_copy(yi.at[pl.ds(base, b_per_w)], yi_v, sem_y)
        la.wait()
        lp.wait()
        ly.wait()
        ca, cp, cy = [], [], []
        for j in range(n_chunks):
            rows = pl.ds(j * CHUNK, CHUNK)
            ca.append(pltpu.async_copy(
                a_hbm.at[ai_v.at[rows]], ar_v.at[rows], sem_a))
            cp.append(pltpu.async_copy(
                p_hbm.at[pi_v.at[rows]], pr_v.at[rows], sem_p))
            cy.append(pltpu.async_copy(
                y_hbm.at[yi_v.at[rows]], yr_v.at[rows], sem_y))
        dst = pl.ds(base, b_per_w)
        for c in ca:
            c.wait()
        wa = pltpu.async_copy(ar_v, out.at[dst, pl.ds(0, EMBED)], sem_a)
        for c in cp:
            c.wait()
        wp = pltpu.async_copy(pr_v, out.at[dst, pl.ds(EMBED, EMBED)], sem_p)
        for c in cy:
            c.wait()
        wy = pltpu.async_copy(yr_v, out.at[dst, pl.ds(2 * EMBED, EMBED)],
                              sem_y)
        wa.wait()
        wp.wait()
        wy.wait()

    return sc_gather3(author_table, publisher_table, year_table,
                      a_idx, p_idx, y_idx)


def kernel(author_idx, publisher_idx, year_idx, author_table,
           publisher_table, year_table):
    a_idx = author_idx.astype(jnp.int32)
    p_idx = publisher_idx.astype(jnp.int32)
    y_idx = year_idx.astype(jnp.int32)
    return _sc_gather_half(a_idx, p_idx, y_idx, author_table,
                           publisher_table, year_table)


# speedup vs baseline: 1.2169x; 1.0017x over previous
import functools

import jax
import jax.numpy as jnp
from jax import lax
from jax.experimental import pallas as pl
from jax.experimental.pallas import tpu as pltpu
from jax.experimental.pallas import tpu_sc as plsc

EMBED = 64
NUM_CORES = 2
NUM_SUBCORES = 16
NUM_WORKERS = NUM_CORES * NUM_SUBCORES
CHUNK = 128


def _sc_gather_concat(a_idx, p_idx, y_idx, author_table, publisher_table,
                      year_table):
    batch = a_idx.shape[0]
    b_per_w = batch // NUM_WORKERS
    n_chunks = b_per_w // CHUNK

    mesh = plsc.VectorSubcoreMesh(core_axis_name="c", subcore_axis_name="s")

    @functools.partial(
        pl.kernel,
        mesh=mesh,
        out_type=jax.ShapeDtypeStruct((batch, 3 * EMBED), jnp.float32),
        scratch_types=[
            pltpu.VMEM((b_per_w,), jnp.int32),
            pltpu.VMEM((b_per_w,), jnp.int32),
            pltpu.VMEM((b_per_w,), jnp.int32),
            pltpu.VMEM((b_per_w, EMBED), jnp.float32),
            pltpu.VMEM((b_per_w, EMBED), jnp.float32),
            pltpu.VMEM((b_per_w, EMBED), jnp.float32),
            pltpu.SemaphoreType.DMA,
            pltpu.SemaphoreType.DMA,
            pltpu.SemaphoreType.DMA,
        ],
        compiler_params=pltpu.CompilerParams(use_tc_tiling_on_sc=False),
    )
    def sc_gather3(a_hbm, p_hbm, y_hbm, ai, pi, yi, out,
                   ai_v, pi_v, yi_v, ar_v, pr_v, yr_v,
                   sem_a, sem_p, sem_y):
        wid = lax.axis_index("s") * NUM_CORES + lax.axis_index("c")
        base = wid * b_per_w
        la = pltpu.async_copy(ai.at[pl.ds(base, b_per_w)], ai_v, sem_a)
        lp = pltpu.async_copy(pi.at[pl.ds(base, b_per_w)], pi_v, sem_p)
        ly = pltpu.async_copy(yi.at[pl.ds(base, b_per_w)], yi_v, sem_y)
        la.wait()
        lp.wait()
        ly.wait()
        ca, cp, cy = [], [], []
        for j in range(n_chunks):
            rows = pl.ds(j * CHUNK, CHUNK)
            ca.append(pltpu.async_copy(
                a_hbm.at[ai_v.at[rows]], ar_v.at[rows], sem_a))
            cp.append(pltpu.async_copy(
                p_hbm.at[pi_v.at[rows]], pr_v.at[rows], sem_p))
            cy.append(pltpu.async_copy(
                y_hbm.at[yi_v.at[rows]], yr_v.at[rows], sem_y))
        dst = pl.ds(base, b_per_w)
        for c in ca:
            c.wait()
        wa = pltpu.async_copy(ar_v, out.at[dst, pl.ds(0, EMBED)], sem_a)
        for c in cp:
            c.wait()
        wp = pltpu.async_copy(pr_v, out.at[dst, pl.ds(EMBED, EMBED)], sem_p)
        for c in cy:
            c.wait()
        wy = pltpu.async_copy(yr_v, out.at[dst, pl.ds(2 * EMBED, EMBED)],
                              sem_y)
        wa.wait()
        wp.wait()
        wy.wait()

    return sc_gather3(author_table, publisher_table, year_table,
                      a_idx, p_idx, y_idx)


def kernel(author_idx, publisher_idx, year_idx, author_table,
           publisher_table, year_table):
    a_idx = author_idx.astype(jnp.int32)
    p_idx = publisher_idx.astype(jnp.int32)
    y_idx = year_idx.astype(jnp.int32)
    return _sc_gather_concat(a_idx, p_idx, y_idx, author_table,
                           publisher_table, year_table)
